# baseline (speedup 1.0000x reference)
import functools

import jax
import jax.numpy as jnp
from jax import lax
from jax.experimental import pallas as pl

D_MODEL = 2048
N_HEADS = 16
N_KV_GROUPS = 4
HPG = N_HEADS // N_KV_GROUPS
D_QK = 128
D_V = 128
BLK = 64
NB = 2048 // BLK
TOP_N = 16
WINDOW = 512
SCALE = 1.0 / (D_QK ** 0.5)
S = 2048
QT = 128
KT = 128
N_QT = S // QT
N_KT = S // KT
WIN_TILES = WINDOW // KT

_f32 = jnp.float32
_bf16 = jnp.bfloat16


def _bf(a):
    return a.astype(_bf16)


def _proj_kernel(x_ref, w_ref, y_ref):
    y_ref[...] = jnp.dot(x_ref[...], w_ref[...], preferred_element_type=_f32)


def _project(x2, w_all):
    N = w_all.shape[1]
    bm, bn = 512, N // 5
    return pl.pallas_call(
        _proj_kernel,
        grid=(S // bm, N // bn),
        in_specs=[
            pl.BlockSpec((bm, D_MODEL), lambda i, j: (i, 0)),
            pl.BlockSpec((D_MODEL, bn), lambda i, j: (0, j)),
        ],
        out_specs=pl.BlockSpec((bm, bn), lambda i, j: (i, j)),
        out_shape=jax.ShapeDtypeStruct((S, N), _f32),
    )(x2, w_all)


def _cmp_kernel(q_ref, k_ref, v_ref, oc_ref, sel_ref):
    k = k_ref[...]
    v = v_ref[...]
    kc = jnp.mean(k.reshape(NB, BLK, D_QK), axis=1)
    vc = jnp.mean(v.reshape(NB, BLK, D_V), axis=1)
    kc_b = _bf(kc)
    vc_b = _bf(vc)
    pos = lax.broadcasted_iota(jnp.int32, (S, NB), 0)
    blk = lax.broadcasted_iota(jnp.int32, (S, NB), 1)
    cmask = ((blk + 1) * BLK - 1 <= pos).astype(_f32)
    imp = jnp.zeros((S, NB), _f32)
    for h in range(HPG):
        qh = q_ref[:, h * D_QK:(h + 1) * D_QK]
        s = lax.dot_general(qh, kc_b, (((1,), (1,)), ((), ())),
                            preferred_element_type=_f32) * SCALE
        s = jnp.where(cmask > 0, s, -1e30)
        m = jnp.max(s, axis=1, keepdims=True)
        m = jnp.where(m > -1e29, m, 0.0)
        p = jnp.exp(s - m) * cmask
        den = jnp.sum(p, axis=1, keepdims=True)
        pn = p / jnp.maximum(den, 1e-9)
        oc_ref[:, h * D_V:(h + 1) * D_V] = jnp.dot(
            _bf(pn), vc_b, preferred_element_type=_f32)
        imp = imp + pn * (1.0 / HPG)
    cols = []
    for n in range(NB):
        cn = imp[:, n:n + 1]
        beats = (imp > cn).astype(jnp.int32)
        if n > 0:
            beats = beats + jnp.where(blk < n, (imp == cn).astype(jnp.int32), 0)
        cols.append(jnp.sum(beats, axis=1, keepdims=True))
    rank = jnp.concatenate(cols, axis=1)
    own = (pos // BLK) == blk
    sel = (rank < TOP_N) | own
    sel_ref[0] = sel.astype(_f32)


def _compress_select(q_bf, k_f32, v_f32):
    return pl.pallas_call(
        _cmp_kernel,
        grid=(N_KV_GROUPS,),
        in_specs=[
            pl.BlockSpec((S, HPG * D_QK), lambda g: (0, g)),
            pl.BlockSpec((S, D_QK), lambda g: (0, g)),
            pl.BlockSpec((S, D_V), lambda g: (0, g)),
        ],
        out_specs=[
            pl.BlockSpec((S, HPG * D_V), lambda g: (0, g)),
            pl.BlockSpec((1, S, NB), lambda g: (g, 0, 0)),
        ],
        out_shape=[
            jax.ShapeDtypeStruct((S, N_HEADS * D_V), _f32),
            jax.ShapeDtypeStruct((N_KV_GROUPS, S, NB), _f32),
        ],
    )(q_bf, k_f32, v_f32)


def _attn_kernel(q_ref, k_ref, v_ref, sel_ref, oc_ref, g_ref, o_ref):
    qt = pl.program_id(1)
    q = q_ref[...]
    sel = sel_ref[0]
    iq = lax.broadcasted_iota(jnp.int32, (QT, KT), 0) + qt * QT
    icol = lax.broadcasted_iota(jnp.int32, (QT, KT), 1)
    col_lo = icol < BLK

    def zero_carry():
        return (jnp.full((QT, 1), -1e30, _f32), jnp.zeros((QT, 1), _f32),
                jnp.zeros((QT, D_V), _f32))

    carry = (*zero_carry(), *zero_carry())

    def upd(mlac, s_masked, mask_f, v_b):
        m0, l0, a0 = mlac
        mn = jnp.maximum(m0, jnp.max(s_masked, axis=1, keepdims=True))
        p = jnp.exp(s_masked - mn) * mask_f
        alpha = jnp.exp(m0 - mn)
        l1 = l0 * alpha + jnp.sum(p, axis=1, keepdims=True)
        a1 = a0 * alpha + jnp.dot(_bf(p), v_b, preferred_element_type=_f32)
        return (mn, l1, a1)

    for kb in range(N_KT):
        sel2a = sel[:, 2 * kb:2 * kb + 1]
        sel2b = sel[:, 2 * kb + 1:2 * kb + 2]
        win_on = qt - kb < WIN_TILES + 1
        sel_any = jnp.maximum(jnp.max(sel2a), jnp.max(sel2b)) > 0

        def tile_body(c, kb=kb, sel2a=sel2a, sel2b=sel2b, win_on=win_on):
            m_s, l_s, a_s, m_w, l_w, a_w = c
            k_t = k_ref[kb * KT:(kb + 1) * KT, :]
            v_t = v_ref[kb * KT:(kb + 1) * KT, :]
            s = lax.dot_general(q, k_t, (((1,), (1,)), ((), ())),
                                preferred_element_type=_f32) * SCALE
            ik = icol + kb * KT
            causal = iq >= ik
            selcol = jnp.where(col_lo, sel2a, sel2b) > 0
            mask_s = selcol & causal
            mf_s = mask_s.astype(_f32)
            m_s, l_s, a_s = upd((m_s, l_s, a_s),
                                jnp.where(mask_s, s, -1e30), mf_s, v_t)

            def win_body(cw):
                mask_w = causal & (iq - ik < WINDOW)
                return upd(cw, jnp.where(mask_w, s, -1e30),
                           mask_w.astype(_f32), v_t)

            m_w, l_w, a_w = lax.cond(win_on, win_body, lambda cw: cw,
                                     (m_w, l_w, a_w))
            return (m_s, l_s, a_s, m_w, l_w, a_w)

        active = (kb <= qt) & (sel_any | win_on)
        carry = lax.cond(active, tile_body, lambda c: c, carry)

    m_s, l_s, a_s, m_w, l_w, a_w = carry
    o_sel = a_s / jnp.maximum(l_s, 1e-9)
    o_win = a_w / jnp.maximum(l_w, 1e-9)
    g = jax.nn.sigmoid(g_ref[0])
    o = g[:, 0:1] * oc_ref[...] + g[:, 1:2] * o_sel + g[:, 2:3] * o_win
    o_ref[...] = _bf(o)


def _attention(q_bf, k_bf, v_bf, sel, out_cmp, glog):
    return pl.pallas_call(
        _attn_kernel,
        grid=(N_HEADS, N_QT),
        in_specs=[
            pl.BlockSpec((QT, D_QK), lambda h, t: (t, h)),
            pl.BlockSpec((S, D_QK), lambda h, t: (0, h // HPG)),
            pl.BlockSpec((S, D_V), lambda h, t: (0, h // HPG)),
            pl.BlockSpec((1, QT, NB), lambda h, t: (h // HPG, t, 0)),
            pl.BlockSpec((QT, D_V), lambda h, t: (t, h)),
            pl.BlockSpec((1, QT, 3), lambda h, t: (h, t, 0)),
        ],
        out_specs=pl.BlockSpec((QT, D_V), lambda h, t: (t, h)),
        out_shape=jax.ShapeDtypeStruct((S, N_HEADS * D_V), _bf16),
    )(q_bf, k_bf, v_bf, sel, out_cmp, glog)


def _out_kernel(o_ref, w_ref, y_ref):
    y_ref[...] = jnp.dot(o_ref[...], w_ref[...], preferred_element_type=_f32)


def _out_proj(o_bf, wo_bf):
    bm, bn = 512, 512
    return pl.pallas_call(
        _out_kernel,
        grid=(S // bm, D_MODEL // bn),
        in_specs=[
            pl.BlockSpec((bm, N_HEADS * D_V), lambda i, j: (i, 0)),
            pl.BlockSpec((N_HEADS * D_V, bn), lambda i, j: (0, j)),
        ],
        out_specs=pl.BlockSpec((bm, bn), lambda i, j: (i, j)),
        out_shape=jax.ShapeDtypeStruct((S, D_MODEL), _f32),
    )(o_bf, wo_bf)


def kernel(x, Wq, Wk, Wv, Wg, Wo):
    x2 = x[0]
    w_all = jnp.concatenate(
        [Wq, Wk, Wv, Wg, jnp.zeros((D_MODEL, 80), _f32)], axis=1)
    y = _project(_bf(x2), _bf(w_all))
    q = y[:, :2048]
    k = y[:, 2048:2560]
    v = y[:, 2560:3072]
    glog = y[:, 3072:3120].reshape(S, N_HEADS, 3).transpose(1, 0, 2)
    q_bf = _bf(q)
    out_cmp, sel = _compress_select(q_bf, k, v)
    o_bf = _attention(q_bf, _bf(k), _bf(v), sel, out_cmp, glog)
    out = _out_proj(o_bf, _bf(Wo))
    return out[None]


# speedup vs baseline: 3.5917x; 3.5917x over previous
import functools

import jax
import jax.numpy as jnp
from jax import lax
from jax.experimental import pallas as pl
from jax.experimental.pallas import tpu as pltpu

D_MODEL = 2048
N_HEADS = 16
N_KV_GROUPS = 4
HPG = N_HEADS // N_KV_GROUPS
D_QK = 128
D_V = 128
BLK = 64
NB = 2048 // BLK
TOP_N = 16
WINDOW = 512
SCALE = 1.0 / (D_QK ** 0.5)
S = 2048
QT = 256
KT = 256

_f32 = jnp.float32
_bf16 = jnp.bfloat16


def _bf(a):
    return a.astype(_bf16)


def _proj_kernel(x_ref, w_ref, y_ref):
    y_ref[...] = jnp.dot(x_ref[...], w_ref[...], preferred_element_type=_f32)


def _project(x2, w_all):
    N = w_all.shape[1]
    bm, bn = 512, N // 5
    return pl.pallas_call(
        _proj_kernel,
        grid=(S // bm, N // bn),
        in_specs=[
            pl.BlockSpec((bm, D_MODEL), lambda i, j: (i, 0)),
            pl.BlockSpec((D_MODEL, bn), lambda i, j: (0, j)),
        ],
        out_specs=pl.BlockSpec((bm, bn), lambda i, j: (i, j)),
        out_shape=jax.ShapeDtypeStruct((S, N), _f32),
    )(x2, w_all)


def _cmp_kernel(q_ref, k_ref, v_ref, oc_ref, sel_ref):
    k = k_ref[...]
    v = v_ref[...]
    kc = jnp.mean(k.reshape(NB, BLK, D_QK), axis=1)
    vc = jnp.mean(v.reshape(NB, BLK, D_V), axis=1)
    kc_b = _bf(kc)
    vc_b = _bf(vc)
    pos = lax.broadcasted_iota(jnp.int32, (S, NB), 0)
    blk = lax.broadcasted_iota(jnp.int32, (S, NB), 1)
    cmask = ((blk + 1) * BLK - 1 <= pos).astype(_f32)
    imp = jnp.zeros((S, NB), _f32)
    for h in range(HPG):
        qh = q_ref[:, h * D_QK:(h + 1) * D_QK]
        s = lax.dot_general(qh, kc_b, (((1,), (1,)), ((), ())),
                            preferred_element_type=_f32) * SCALE
        s = jnp.where(cmask > 0, s, -1e30)
        m = jnp.max(s, axis=1, keepdims=True)
        m = jnp.where(m > -1e29, m, 0.0)
        p = jnp.exp(s - m) * cmask
        den = jnp.sum(p, axis=1, keepdims=True)
        pn = p / jnp.maximum(den, 1e-9)
        oc_ref[:, h * D_V:(h + 1) * D_V] = jnp.dot(
            _bf(pn), vc_b, preferred_element_type=_f32)
        imp = imp + pn * (1.0 / HPG)
    cols = []
    for n in range(NB):
        cn = imp[:, n:n + 1]
        beats = (imp > cn).astype(jnp.int32)
        if n > 0:
            beats = beats + jnp.where(blk < n, (imp == cn).astype(jnp.int32), 0)
        cols.append(jnp.sum(beats, axis=1, keepdims=True))
    rank = jnp.concatenate(cols, axis=1)
    own = (pos // BLK) == blk
    sel = (rank < TOP_N) | own
    sel_ref[0] = sel.astype(_f32)


def _compress_select(q_bf, k_f32, v_f32):
    return pl.pallas_call(
        _cmp_kernel,
        grid=(N_KV_GROUPS,),
        in_specs=[
            pl.BlockSpec((S, HPG * D_QK), lambda g: (0, g)),
            pl.BlockSpec((S, D_QK), lambda g: (0, g)),
            pl.BlockSpec((S, D_V), lambda g: (0, g)),
        ],
        out_specs=[
            pl.BlockSpec((S, HPG * D_V), lambda g: (0, g)),
            pl.BlockSpec((1, S, NB), lambda g: (g, 0, 0)),
        ],
        out_shape=[
            jax.ShapeDtypeStruct((S, N_HEADS * D_V), _f32),
            jax.ShapeDtypeStruct((N_KV_GROUPS, S, NB), _f32),
        ],
    )(q_bf, k_f32, v_f32)


def _upd(mla, s_masked, v_b):
    m0, l0, a0 = mla
    mn = jnp.maximum(m0, jnp.max(s_masked, axis=1, keepdims=True))
    p = jnp.exp(s_masked - mn)
    alpha = jnp.exp(m0 - mn)
    l1 = l0 * alpha + jnp.sum(p, axis=1, keepdims=True)
    a1 = a0 * alpha + jnp.dot(_bf(p), v_b, preferred_element_type=_f32)
    return (mn, l1, a1)


def _attn_kernel(q_ref, kt_ref, v_ref, sel_ref, exp_ref, oc_ref, g_ref,
                 o_ref, selm_ref):
    qt = pl.program_id(1)
    q = q_ref[...]
    selm_ref[...] = jnp.dot(_bf(sel_ref[0]), exp_ref[...],
                            preferred_element_type=_f32)
    iq = lax.broadcasted_iota(jnp.int32, (QT, KT), 0)
    ic = lax.broadcasted_iota(jnp.int32, (QT, KT), 1)
    rel = iq - ic
    causal = rel >= 0
    neg_inf = jnp.float32(-jnp.inf)

    def zero3():
        return (jnp.full((QT, 1), -1e30, _f32), jnp.zeros((QT, 1), _f32),
                jnp.zeros((QT, D_V), _f32))

    carry = tuple(x for _ in range(2 * HPG) for x in zero3())

    def tile_body(kb, c, with_win):
        c = list(c)
        k_t = kt_ref[0, :, pl.ds(kb * KT, KT)]
        v_t = v_ref[0, pl.ds(kb * KT, KT), :]
        selb = selm_ref[:, pl.ds(kb * KT, KT)] > 0
        win_cut = WINDOW - (qt - kb) * KT
        for h in range(HPG):
            s = jnp.dot(q[:, h * D_QK:(h + 1) * D_QK], k_t,
                        preferred_element_type=_f32) * SCALE
            c[3 * h:3 * h + 3] = _upd(tuple(c[3 * h:3 * h + 3]),
                                      jnp.where(selb, s, neg_inf), v_t)
            if with_win:
                j = 3 * (HPG + h)
                c[j:j + 3] = _upd(tuple(c[j:j + 3]),
                                  jnp.where(rel < win_cut, s, neg_inf), v_t)
        return tuple(c)

    t0 = jnp.maximum(qt - WINDOW // KT, 0)
    carry = lax.fori_loop(0, t0, lambda kb, c: tile_body(kb, c, False), carry)
    carry = lax.fori_loop(t0, qt, lambda kb, c: tile_body(kb, c, True), carry)
    c = list(carry)
    k_t = kt_ref[0, :, pl.ds(qt * KT, KT)]
    v_t = v_ref[0, pl.ds(qt * KT, KT), :]
    selb = (selm_ref[:, pl.ds(qt * KT, KT)] > 0) & causal
    for h in range(HPG):
        s = jnp.dot(q[:, h * D_QK:(h + 1) * D_QK], k_t,
                    preferred_element_type=_f32) * SCALE
        c[3 * h:3 * h + 3] = _upd(tuple(c[3 * h:3 * h + 3]),
                                  jnp.where(selb, s, neg_inf), v_t)
        j = 3 * (HPG + h)
        c[j:j + 3] = _upd(tuple(c[j:j + 3]), jnp.where(causal, s, neg_inf),
                          v_t)
    g = jax.nn.sigmoid(g_ref[0])
    for h in range(HPG):
        _, l_s, a_s = c[3 * h:3 * h + 3]
        _, l_w, a_w = c[3 * (HPG + h):3 * (HPG + h) + 3]
        o_sel = a_s / jnp.maximum(l_s, 1e-9)
        o_win = a_w / jnp.maximum(l_w, 1e-9)
        o = (g[:, 3 * h:3 * h + 1] * oc_ref[:, h * D_V:(h + 1) * D_V]
             + g[:, 3 * h + 1:3 * h + 2] * o_sel
             + g[:, 3 * h + 2:3 * h + 3] * o_win)
        o_ref[:, h * D_V:(h + 1) * D_V] = _bf(o)


def _attention(q_bf, kt_bf, v_bf, sel, expand_bf, out_cmp, glog):
    return pl.pallas_call(
        _attn_kernel,
        grid=(N_KV_GROUPS, S // QT),
        in_specs=[
            pl.BlockSpec((QT, HPG * D_QK), lambda g, t: (t, g)),
            pl.BlockSpec((1, D_QK, S), lambda g, t: (g, 0, 0)),
            pl.BlockSpec((1, S, D_V), lambda g, t: (g, 0, 0)),
            pl.BlockSpec((1, QT, NB), lambda g, t: (g, t, 0)),
            pl.BlockSpec((NB, S), lambda g, t: (0, 0)),
            pl.BlockSpec((QT, HPG * D_V), lambda g, t: (t, g)),
            pl.BlockSpec((1, QT, 3 * HPG), lambda g, t: (g, t, 0)),
        ],
        out_specs=pl.BlockSpec((QT, HPG * D_V), lambda g, t: (t, g)),
        out_shape=jax.ShapeDtypeStruct((S, N_HEADS * D_V), _bf16),
        scratch_shapes=[pltpu.VMEM((QT, S), _f32)],
    )(q_bf, kt_bf, v_bf, sel, expand_bf, out_cmp, glog)


def _out_kernel(o_ref, w_ref, y_ref):
    y_ref[...] = jnp.dot(o_ref[...], w_ref[...], preferred_element_type=_f32)


def _out_proj(o_bf, wo_bf):
    bm, bn = 512, 512
    return pl.pallas_call(
        _out_kernel,
        grid=(S // bm, D_MODEL // bn),
        in_specs=[
            pl.BlockSpec((bm, N_HEADS * D_V), lambda i, j: (i, 0)),
            pl.BlockSpec((N_HEADS * D_V, bn), lambda i, j: (0, j)),
        ],
        out_specs=pl.BlockSpec((bm, bn), lambda i, j: (i, j)),
        out_shape=jax.ShapeDtypeStruct((S, D_MODEL), _f32),
    )(o_bf, wo_bf)


def kernel(x, Wq, Wk, Wv, Wg, Wo):
    x2 = x[0]
    w_all = jnp.concatenate(
        [Wq, Wk, Wv, Wg, jnp.zeros((D_MODEL, 80), _f32)], axis=1)
    y = _project(_bf(x2), _bf(w_all))
    q = y[:, :2048]
    k = y[:, 2048:2560]
    v = y[:, 2560:3072]
    glog = y[:, 3072:3120].reshape(S, N_KV_GROUPS, 3 * HPG).transpose(1, 0, 2)
    q_bf = _bf(q)
    out_cmp, sel = _compress_select(q_bf, k, v)
    kt_bf = _bf(k).reshape(S, N_KV_GROUPS, D_QK).transpose(1, 2, 0)
    v_bf = _bf(v).reshape(S, N_KV_GROUPS, D_V).transpose(1, 0, 2)
    expand_bf = (jnp.arange(S)[None, :] // BLK
                 == jnp.arange(NB)[:, None]).astype(_bf16)
    o_bf = _attention(q_bf, kt_bf, v_bf, sel, expand_bf, out_cmp, glog)
    out = _out_proj(o_bf, _bf(Wo))
    return out[None]
